# baseline (speedup 1.0000x reference)
import functools

import jax
import jax.numpy as jnp
from jax import lax
from jax.experimental import pallas as pl
from jax.experimental.pallas import tpu as pltpu
from jax.experimental.pallas import tpu_sc as plsc

N = 10000
E = 160000
D = 256
NC = 2
NS = 16
L = 16
NPAD = 10240
EPT = E // NS
VPT = EPT // L
SLICE = NPAD // (NC * NS)
CH = 64
NCH = SLICE // CH

_mesh = plsc.VectorSubcoreMesh(core_axis_name="c", subcore_axis_name="s")


@functools.partial(
    pl.kernel,
    out_type=jax.ShapeDtypeStruct((NPAD, D), jnp.float32),
    mesh=_mesh,
    scratch_types=[
        pltpu.VMEM((EPT,), jnp.int32),
        pltpu.VMEM((EPT,), jnp.int32),
        pltpu.VMEM((NPAD,), jnp.int32),
        pltpu.VMEM_SHARED((NS * NPAD,), jnp.int32),
        pltpu.VMEM((NS * SLICE,), jnp.int32),
        pltpu.VMEM((SLICE,), jnp.int32),
        pltpu.VMEM((SLICE,), jnp.int32),
        pltpu.VMEM((CH, D), jnp.float32),
        pltpu.VMEM((CH, D), jnp.float32),
        pltpu.VMEM((L, D), jnp.float32),
        pltpu.SemaphoreType.DMA,
        pltpu.SemaphoreType.DMA,
    ],
    compiler_params=pltpu.CompilerParams(needs_layout_passes=False),
)
def _agg_kernel(dst_hbm, src_hbm, x_hbm, agg_hbm,
                dst_v, src_v, p_v, shared, comb, res_v, neg_v, g0, g1,
                zrow_v, sem0, sem1):
    c = lax.axis_index("c")
    sid = lax.axis_index("s")
    base = sid * EPT
    pltpu.sync_copy(dst_hbm.at[pl.ds(base, EPT)], dst_v)
    pltpu.sync_copy(src_hbm.at[pl.ds(base, EPT)], src_v)

    neg1 = jnp.full((L,), -1, jnp.int32)

    def init_body(i, _):
        p_v[pl.ds(i * L, L)] = neg1
        return 0

    lax.fori_loop(0, NPAD // L, init_body, 0)

    lane = lax.iota(jnp.int32, L)
    roll1 = (lane + 1) % L
    is_last_lane = lane == (L - 1)

    UNROLL = 5

    def edge_body(i, _):
        sorted_runs = []
        for u in range(UNROLL):
            off = (i * UNROLL + u) * L
            d = dst_v[pl.ds(off, L)]
            s = src_v[pl.ds(off, L)]
            key = jnp.left_shift(d, 4) | lane
            sorted_runs.append(plsc.sort_key_val(key, s))
        for sk, sv in sorted_runs:
            sd = jnp.right_shift(sk, 4)
            nxt = sd.at[roll1].get(mode="promise_in_bounds")
            last = is_last_lane | (nxt != sd)
            plsc.store_scatter(p_v, [sd], sv, mask=last)
        return 0

    lax.fori_loop(0, VPT // UNROLL, edge_body, 0)

    pltpu.sync_copy(p_v, shared.at[pl.ds(sid * NPAD, NPAD)])
    plsc.subcore_barrier()

    q = sid * NC + c
    for t in range(NS):
        pltpu.sync_copy(
            shared.at[pl.ds(t * NPAD + q * SLICE, SLICE)],
            comb.at[pl.ds(t * SLICE, SLICE)],
        )

    def comb_body(v, cnt):
        acc = jnp.full((L,), -1, jnp.int32)
        for t in range(NS):
            val = comb[pl.ds(t * SLICE + v * L, L)]
            acc = jnp.where(val >= 0, val, acc)
        neg_v[pl.ds(v * L, L)] = acc
        res_v[pl.ds(v * L, L)] = jnp.maximum(acc, 0)
        return cnt + jnp.sum((acc < 0).astype(jnp.int32), axis=0)

    n_sent = lax.fori_loop(0, SLICE // L, comb_body, jnp.int32(0))

    rowbase = q * SLICE
    bufs, sems = (g0, g1), (sem0, sem1)
    handles = [None] * NCH

    def start(ch):
        return pltpu.async_copy(
            x_hbm.at[res_v.at[pl.ds(ch * CH, CH)]], bufs[ch % 2], sems[ch % 2]
        )

    handles[0] = start(0)
    for ch in range(NCH):
        if ch + 1 < NCH:
            handles[ch + 1] = start(ch + 1)
        handles[ch].wait()
        pltpu.sync_copy(bufs[ch % 2], agg_hbm.at[pl.ds(rowbase + ch * CH, CH)])

    @pl.when(n_sent > 0)
    def _fix():
        zero16 = jnp.zeros((L,), jnp.float32)

        def zb_body(i, _):
            zrow_v[i // (D // L), pl.ds((i % (D // L)) * L, L)] = zero16
            return 0

        lax.fori_loop(0, L * (D // L), zb_body, 0)

        def fix_body(v, _):
            a = neg_v[pl.ds(v * L, L)]
            hits = jnp.sum((a < 0).astype(jnp.int32), axis=0)

            @pl.when(hits > 0)
            def _():
                rows = jnp.where(a < 0, rowbase + v * L + lane, NPAD - 1)
                pltpu.async_copy(zrow_v, agg_hbm.at[rows], sem0).wait()

            return 0

        lax.fori_loop(0, SLICE // L, fix_body, 0)


def _mm_body(x_ref, a_ref, w1_ref, w2_ref, bias_ref, o_ref):
    acc = jnp.dot(x_ref[...], w1_ref[...], preferred_element_type=jnp.float32)
    acc = acc + jnp.dot(a_ref[...], w2_ref[...], preferred_element_type=jnp.float32)
    o_ref[...] = acc + bias_ref[...]


_MM_BLOCK = 400
_mm_call = pl.pallas_call(
    _mm_body,
    grid=(N // _MM_BLOCK,),
    in_specs=[
        pl.BlockSpec((_MM_BLOCK, D), lambda i: (i, 0)),
        pl.BlockSpec((_MM_BLOCK, D), lambda i: (i, 0)),
        pl.BlockSpec((D, D), lambda i: (0, 0)),
        pl.BlockSpec((D, D), lambda i: (0, 0)),
        pl.BlockSpec((1, D), lambda i: (0, 0)),
    ],
    out_specs=pl.BlockSpec((_MM_BLOCK, D), lambda i: (i, 0)),
    out_shape=jax.ShapeDtypeStruct((N, D), jnp.float32),
)


def kernel(x, edge_index, W, b):
    dst = edge_index[0]
    src = edge_index[1]
    w1t = W[:, :D].T
    w2t = W[:, D:].T
    bias = b.reshape(1, D)

    agg = _agg_kernel(dst, src, x)
    return _mm_call(x, agg, w1t, w2t, bias)


# speedup vs baseline: 1.1465x; 1.1465x over previous
import functools

import jax
import jax.numpy as jnp
from jax import lax
from jax.experimental import pallas as pl
from jax.experimental.pallas import tpu as pltpu
from jax.experimental.pallas import tpu_sc as plsc

N = 10000
E = 160000
D = 256
NC = 2
NS = 16
L = 16
NPAD = 10240
EPT = E // NS
VPT = EPT // L
SLICE = NPAD // (NC * NS)
CH = 80
NCH = SLICE // CH

_mesh = plsc.VectorSubcoreMesh(core_axis_name="c", subcore_axis_name="s")


@functools.partial(
    pl.kernel,
    out_type=jax.ShapeDtypeStruct((NPAD, D), jnp.float32),
    mesh=_mesh,
    scratch_types=[
        pltpu.VMEM((EPT,), jnp.int32),
        pltpu.VMEM((EPT,), jnp.int32),
        pltpu.VMEM((NPAD,), jnp.int32),
        pltpu.VMEM_SHARED((NS * NPAD,), jnp.int32),
        pltpu.VMEM((NS * SLICE,), jnp.int32),
        pltpu.VMEM((SLICE,), jnp.int32),
        pltpu.VMEM((SLICE,), jnp.int32),
        pltpu.VMEM((CH, D), jnp.float32),
        pltpu.VMEM((CH, D), jnp.float32),
        pltpu.VMEM((L, D), jnp.float32),
        pltpu.SemaphoreType.DMA,
        pltpu.SemaphoreType.DMA,
        pltpu.SemaphoreType.DMA,
        pltpu.SemaphoreType.DMA,
    ],
    compiler_params=pltpu.CompilerParams(needs_layout_passes=False),
)
def _agg_kernel(dst_hbm, src_hbm, x_hbm, agg_hbm,
                dst_v, src_v, p_v, shared, comb, res_v, neg_v, g0, g1,
                zrow_v, sem0, sem1, sem2, sem3):
    c = lax.axis_index("c")
    sid = lax.axis_index("s")
    base = sid * EPT
    pltpu.sync_copy(dst_hbm.at[pl.ds(base, EPT)], dst_v)
    pltpu.sync_copy(src_hbm.at[pl.ds(base, EPT)], src_v)

    neg1 = jnp.full((L,), -1, jnp.int32)

    def init_body(i, _):
        p_v[pl.ds(i * L, L)] = neg1
        return 0

    lax.fori_loop(0, NPAD // L, init_body, 0)

    lane = lax.iota(jnp.int32, L)
    roll1 = (lane + 1) % L
    is_last_lane = lane == (L - 1)

    UNROLL = 5

    def edge_body(i, _):
        sorted_runs = []
        for u in range(UNROLL):
            off = (i * UNROLL + u) * L
            d = dst_v[pl.ds(off, L)]
            s = src_v[pl.ds(off, L)]
            key = jnp.left_shift(d, 4) | lane
            sorted_runs.append(plsc.sort_key_val(key, s))
        for sk, sv in sorted_runs:
            sd = jnp.right_shift(sk, 4)
            nxt = sd.at[roll1].get(mode="promise_in_bounds")
            last = is_last_lane | (nxt != sd)
            plsc.store_scatter(p_v, [sd], sv, mask=last)
        return 0

    lax.fori_loop(0, VPT // UNROLL, edge_body, 0)

    pltpu.sync_copy(p_v, shared.at[pl.ds(sid * NPAD, NPAD)])
    plsc.subcore_barrier()

    q = sid * NC + c
    for t in range(NS):
        pltpu.sync_copy(
            shared.at[pl.ds(t * NPAD + q * SLICE, SLICE)],
            comb.at[pl.ds(t * SLICE, SLICE)],
        )

    def comb_body(v, cnt):
        acc = jnp.full((L,), -1, jnp.int32)
        for t in range(NS):
            val = comb[pl.ds(t * SLICE + v * L, L)]
            acc = jnp.where(val >= 0, val, acc)
        neg_v[pl.ds(v * L, L)] = acc
        res_v[pl.ds(v * L, L)] = jnp.maximum(acc, 0)
        return cnt + jnp.sum((acc < 0).astype(jnp.int32), axis=0)

    n_sent = lax.fori_loop(0, SLICE // L, comb_body, jnp.int32(0))

    rowbase = q * SLICE
    bufs, gsems, ssems = (g0, g1), (sem0, sem1), (sem2, sem3)
    gh = [None] * NCH
    sh = [None, None]

    def start(ch):
        return pltpu.async_copy(
            x_hbm.at[res_v.at[pl.ds(ch * CH, CH)]], bufs[ch % 2], gsems[ch % 2]
        )

    gh[0] = start(0)
    for ch in range(NCH):
        if ch + 1 < NCH:
            if sh[(ch + 1) % 2] is not None:
                sh[(ch + 1) % 2].wait()
            gh[ch + 1] = start(ch + 1)
        gh[ch].wait()
        sh[ch % 2] = pltpu.async_copy(
            bufs[ch % 2], agg_hbm.at[pl.ds(rowbase + ch * CH, CH)], ssems[ch % 2]
        )
    for b in range(2):
        if sh[b] is not None:
            sh[b].wait()

    @pl.when(n_sent > 0)
    def _fix():
        zero16 = jnp.zeros((L,), jnp.float32)

        def zb_body(i, _):
            zrow_v[i // (D // L), pl.ds((i % (D // L)) * L, L)] = zero16
            return 0

        lax.fori_loop(0, L * (D // L), zb_body, 0)

        def fix_body(v, _):
            a = neg_v[pl.ds(v * L, L)]
            hits = jnp.sum((a < 0).astype(jnp.int32), axis=0)

            @pl.when(hits > 0)
            def _():
                rows = jnp.where(a < 0, rowbase + v * L + lane, NPAD - 1)
                pltpu.async_copy(zrow_v, agg_hbm.at[rows], sem0).wait()

            return 0

        lax.fori_loop(0, SLICE // L, fix_body, 0)


def _mm_body(x_ref, a_ref, w1_ref, w2_ref, bias_ref, o_ref):
    acc = jnp.dot(x_ref[...], w1_ref[...], preferred_element_type=jnp.float32)
    acc = acc + jnp.dot(a_ref[...], w2_ref[...], preferred_element_type=jnp.float32)
    o_ref[...] = acc + bias_ref[...]


_MM_BLOCK = 2000
_mm_call = pl.pallas_call(
    _mm_body,
    grid=(N // _MM_BLOCK,),
    in_specs=[
        pl.BlockSpec((_MM_BLOCK, D), lambda i: (i, 0)),
        pl.BlockSpec((_MM_BLOCK, D), lambda i: (i, 0)),
        pl.BlockSpec((D, D), lambda i: (0, 0)),
        pl.BlockSpec((D, D), lambda i: (0, 0)),
        pl.BlockSpec((1, D), lambda i: (0, 0)),
    ],
    out_specs=pl.BlockSpec((_MM_BLOCK, D), lambda i: (i, 0)),
    out_shape=jax.ShapeDtypeStruct((N, D), jnp.float32),
)


def kernel(x, edge_index, W, b):
    dst = edge_index[0]
    src = edge_index[1]
    w1t = W[:, :D].T
    w2t = W[:, D:].T
    bias = b.reshape(1, D)

    agg = _agg_kernel(dst, src, x)
    return _mm_call(x, agg, w1t, w2t, bias)


# speedup vs baseline: 1.2045x; 1.0506x over previous
import functools

import jax
import jax.numpy as jnp
from jax import lax
from jax.experimental import pallas as pl
from jax.experimental.pallas import tpu as pltpu
from jax.experimental.pallas import tpu_sc as plsc

N = 10000
E = 160000
D = 256
NC = 2
NS = 16
L = 16
NPAD = 10240
EPT = E // NS
VPT = EPT // L
SLICE = NPAD // (NC * NS)
CH = 80
NCH = SLICE // CH

_mesh = plsc.VectorSubcoreMesh(core_axis_name="c", subcore_axis_name="s")


@functools.partial(
    pl.kernel,
    out_type=jax.ShapeDtypeStruct((NPAD, D), jnp.float32),
    mesh=_mesh,
    scratch_types=[
        pltpu.VMEM((EPT,), jnp.int32),
        pltpu.VMEM((EPT,), jnp.int32),
        pltpu.VMEM((NPAD,), jnp.int32),
        pltpu.VMEM_SHARED((NS * NPAD,), jnp.int32),
        pltpu.VMEM((NS * SLICE,), jnp.int32),
        pltpu.VMEM((SLICE,), jnp.int32),
        pltpu.VMEM((SLICE,), jnp.int32),
        pltpu.VMEM((CH, D), jnp.float32),
        pltpu.VMEM((CH, D), jnp.float32),
        pltpu.VMEM((L, D), jnp.float32),
        pltpu.SemaphoreType.DMA,
        pltpu.SemaphoreType.DMA,
        pltpu.SemaphoreType.DMA,
        pltpu.SemaphoreType.DMA,
    ],
    compiler_params=pltpu.CompilerParams(needs_layout_passes=False),
)
def _agg_kernel(dst_hbm, src_hbm, x_hbm, agg_hbm,
                dst_v, src_v, p_v, shared, comb, res_v, neg_v, g0, g1,
                zrow_v, sem0, sem1, sem2, sem3):
    c = lax.axis_index("c")
    sid = lax.axis_index("s")
    base = sid * EPT
    eh0 = pltpu.async_copy(dst_hbm.at[pl.ds(base, EPT)], dst_v, sem0)
    eh1 = pltpu.async_copy(src_hbm.at[pl.ds(base, EPT)], src_v, sem1)

    neg1 = jnp.full((L,), -1, jnp.int32)

    def init_body(i, _):
        p_v[pl.ds(i * L, L)] = neg1
        return 0

    lax.fori_loop(0, NPAD // L, init_body, 0)
    eh0.wait()
    eh1.wait()

    lane = lax.iota(jnp.int32, L)
    roll1 = (lane + 1) % L
    is_last_lane = lane == (L - 1)

    UNROLL = 5

    def edge_body(i, _):
        sorted_runs = []
        for u in range(UNROLL):
            off = (i * UNROLL + u) * L
            d = dst_v[pl.ds(off, L)]
            s = src_v[pl.ds(off, L)]
            key = jnp.left_shift(d, 4) | lane
            sorted_runs.append(plsc.sort_key_val(key, s))
        for sk, sv in sorted_runs:
            sd = jnp.right_shift(sk, 4)
            nxt = sd.at[roll1].get(mode="promise_in_bounds")
            last = is_last_lane | (nxt != sd)
            plsc.store_scatter(p_v, [sd], sv, mask=last)
        return 0

    lax.fori_loop(0, VPT // UNROLL, edge_body, 0)

    pltpu.sync_copy(p_v, shared.at[pl.ds(sid * NPAD, NPAD)])
    plsc.subcore_barrier()

    q = sid * NC + c
    comb_hs = [
        pltpu.async_copy(
            shared.at[pl.ds(t * NPAD + q * SLICE, SLICE)],
            comb.at[pl.ds(t * SLICE, SLICE)],
            sem0,
        )
        for t in range(NS)
    ]
    for h in comb_hs:
        h.wait()

    def comb_body(v, cnt):
        acc = jnp.full((L,), -1, jnp.int32)
        for t in range(NS):
            val = comb[pl.ds(t * SLICE + v * L, L)]
            acc = jnp.where(val >= 0, val, acc)
        neg_v[pl.ds(v * L, L)] = acc
        res_v[pl.ds(v * L, L)] = jnp.maximum(acc, 0)
        return cnt + jnp.sum((acc < 0).astype(jnp.int32), axis=0)

    n_sent = lax.fori_loop(0, SLICE // L, comb_body, jnp.int32(0))

    rowbase = q * SLICE
    bufs, gsems, ssems = (g0, g1), (sem0, sem1), (sem2, sem3)
    gh = [None] * NCH
    sh = [None, None]

    def start(ch):
        return pltpu.async_copy(
            x_hbm.at[res_v.at[pl.ds(ch * CH, CH)]], bufs[ch % 2], gsems[ch % 2]
        )

    gh[0] = start(0)
    for ch in range(NCH):
        if ch + 1 < NCH:
            if sh[(ch + 1) % 2] is not None:
                sh[(ch + 1) % 2].wait()
            gh[ch + 1] = start(ch + 1)
        gh[ch].wait()
        sh[ch % 2] = pltpu.async_copy(
            bufs[ch % 2], agg_hbm.at[pl.ds(rowbase + ch * CH, CH)], ssems[ch % 2]
        )
    for b in range(2):
        if sh[b] is not None:
            sh[b].wait()

    @pl.when(n_sent > 0)
    def _fix():
        zero16 = jnp.zeros((L,), jnp.float32)

        def zb_body(i, _):
            zrow_v[i // (D // L), pl.ds((i % (D // L)) * L, L)] = zero16
            return 0

        lax.fori_loop(0, L * (D // L), zb_body, 0)

        def fix_body(v, _):
            a = neg_v[pl.ds(v * L, L)]
            hits = jnp.sum((a < 0).astype(jnp.int32), axis=0)

            @pl.when(hits > 0)
            def _():
                rows = jnp.where(a < 0, rowbase + v * L + lane, NPAD - 1)
                pltpu.async_copy(zrow_v, agg_hbm.at[rows], sem0).wait()

            return 0

        lax.fori_loop(0, SLICE // L, fix_body, 0)


def _mm_body(x_ref, a_ref, w1_ref, w2_ref, bias_ref, o_ref):
    acc = jnp.dot(x_ref[...], w1_ref[...], preferred_element_type=jnp.float32)
    acc = acc + jnp.dot(a_ref[...], w2_ref[...], preferred_element_type=jnp.float32)
    o_ref[...] = acc + bias_ref[...]


_MM_BLOCK = 2000
_mm_call = pl.pallas_call(
    _mm_body,
    grid=(N // _MM_BLOCK,),
    in_specs=[
        pl.BlockSpec((_MM_BLOCK, D), lambda i: (i, 0)),
        pl.BlockSpec((_MM_BLOCK, D), lambda i: (i, 0)),
        pl.BlockSpec((D, D), lambda i: (0, 0)),
        pl.BlockSpec((D, D), lambda i: (0, 0)),
        pl.BlockSpec((1, D), lambda i: (0, 0)),
    ],
    out_specs=pl.BlockSpec((_MM_BLOCK, D), lambda i: (i, 0)),
    out_shape=jax.ShapeDtypeStruct((N, D), jnp.float32),
)


def kernel(x, edge_index, W, b):
    dst = edge_index[0]
    src = edge_index[1]
    w1t = W[:, :D].T
    w2t = W[:, D:].T
    bias = b.reshape(1, D)

    agg = _agg_kernel(dst, src, x)
    return _mm_call(x, agg, w1t, w2t, bias)
